# baseline (speedup 1.0000x reference)
import jax
import jax.numpy as jnp
from jax import lax
from jax.experimental import pallas as pl
from jax.experimental.pallas import tpu as pltpu
from jax.experimental.pallas import tpu_sc as plsc

N_USERS = 100000
N_ITEMS = 900000
EMB = 32
REGS = 1e-5
B = 16384

NC = 2
NS = 16
NW = NC * NS
PB = B // NW
CHUNK = 128
NCH = PB // CHUNK
QT = 262144
RG = QT // 128


def _repack_body(t0, t1, t2, t3, out_ref):
    for j, t in enumerate((t0, t1, t2, t3)):
        out_ref[:, pl.ds(j * EMB, EMB)] = jnp.transpose(t[...])


def _repack(tbl_t):
    specs = [
        pl.BlockSpec((EMB, 128), lambda i, j=j: (0, i + j * RG))
        for j in range(4)
    ]
    return pl.pallas_call(
        _repack_body,
        grid=(RG,),
        in_specs=specs,
        out_specs=pl.BlockSpec((128, 128), lambda i: (i, 0)),
        out_shape=jax.ShapeDtypeStruct((QT, 128), jnp.float32),
    )(tbl_t, tbl_t, tbl_t, tbl_t)


def _sc_body(tbl_hbm, u_idx_hbm, p_idx_hbm, n_idx_hbm,
             sc_hbm, sq_hbm,
             u_idx_v, p_idx_v, n_idx_v,
             uo_v, po_v, no_v,
             u_blk, p_blk, n_blk,
             sc_v, sq_v, sem):
    wid = lax.axis_index("s") * NC + lax.axis_index("c")

    pltpu.sync_copy(u_idx_hbm.at[wid], u_idx_v)
    pltpu.sync_copy(p_idx_hbm.at[wid], p_idx_v)
    pltpu.sync_copy(n_idx_hbm.at[wid], n_idx_v)
    for src, off in ((u_idx_v, uo_v), (p_idx_v, po_v), (n_idx_v, no_v)):
        for j in range(NCH):
            for v in range(CHUNK // 16):
                x = src[j, pl.ds(v * 16, 16)]
                off[j, pl.ds(v * 16, 16)] = lax.shift_right_logical(x, 18) * 32
                src[j, pl.ds(v * 16, 16)] = x & (QT - 1)

    sq = jnp.zeros((16,), jnp.float32)
    for p in range(NCH):
        copies = []
        for idx_v, blk in ((u_idx_v, u_blk), (p_idx_v, p_blk), (n_idx_v, n_blk)):
            copies.append(pltpu.async_copy(
                tbl_hbm.at[idx_v.at[p]], blk, sem))
        for c in copies:
            c.wait()

        def group(g, sq):
            lanes = pl.ds(g * 16, 16)
            ku = lax.iota(jnp.int32, 16) + g * 16
            offu = uo_v[p, lanes]
            offp = po_v[p, lanes]
            offn = no_v[p, lanes]

            def dim(d, carry):
                acc, sq = carry
                vu = plsc.load_gather(u_blk, [ku, offu + d])
                vp = plsc.load_gather(p_blk, [ku, offp + d])
                vn = plsc.load_gather(n_blk, [ku, offn + d])
                return (acc + vu * (vp - vn),
                        sq + vu * vu + vp * vp + vn * vn)

            acc, sq = lax.fori_loop(
                0, EMB, dim, (jnp.zeros((16,), jnp.float32), sq))
            sc_v[p, lanes] = acc
            return sq

        sq = lax.fori_loop(0, CHUNK // 16, group, sq)

    sq_v[...] = sq
    pltpu.sync_copy(sc_v, sc_hbm.at[wid])
    pltpu.sync_copy(sq_v, sq_hbm.at[wid])


def _sc_call(tbl, u_idx, p_idx, n_idx):
    mesh = plsc.VectorSubcoreMesh(core_axis_name="c", subcore_axis_name="s")
    return pl.kernel(
        _sc_body,
        out_type=(
            jax.ShapeDtypeStruct((NW, NCH, CHUNK), jnp.float32),
            jax.ShapeDtypeStruct((NW, 16), jnp.float32),
        ),
        mesh=mesh,
        compiler_params=pltpu.CompilerParams(
            use_tc_tiling_on_sc=True, needs_layout_passes=False),
        scratch_types=[
            pltpu.VMEM((NCH, CHUNK), jnp.int32),
            pltpu.VMEM((NCH, CHUNK), jnp.int32),
            pltpu.VMEM((NCH, CHUNK), jnp.int32),
            pltpu.VMEM((NCH, CHUNK), jnp.int32),
            pltpu.VMEM((NCH, CHUNK), jnp.int32),
            pltpu.VMEM((NCH, CHUNK), jnp.int32),
            pltpu.VMEM((CHUNK, 128), jnp.float32),
            pltpu.VMEM((CHUNK, 128), jnp.float32),
            pltpu.VMEM((CHUNK, 128), jnp.float32),
            pltpu.VMEM((NCH, CHUNK), jnp.float32),
            pltpu.VMEM((16,), jnp.float32),
            pltpu.SemaphoreType.DMA,
        ],
    )(tbl, u_idx, p_idx, n_idx)


def _tc_epilogue_body(sc_ref, sq_ref, bpr_ref, reg_ref):
    d = sc_ref[...]
    logsig = -jnp.log1p(jnp.exp(-d))
    bpr_ref[...] = jnp.full((1, 1), -jnp.mean(logsig), jnp.float32)
    reg_ref[...] = jnp.full((1, 1), REGS * 0.5 * jnp.sum(sq_ref[...]),
                            jnp.float32)


def _tc_epilogue(sc, sq):
    return pl.pallas_call(
        _tc_epilogue_body,
        out_shape=(
            jax.ShapeDtypeStruct((1, 1), jnp.float32),
            jax.ShapeDtypeStruct((1, 1), jnp.float32),
        ),
    )(sc, sq)


@jax.jit
def kernel(user, pos_item, neg_item, table):
    tbl_t = table[:1000000].T
    tbl = _repack(tbl_t)
    u_idx = user.reshape(NW, NCH, CHUNK)
    p_idx = pos_item.reshape(NW, NCH, CHUNK)
    n_idx = neg_item.reshape(NW, NCH, CHUNK)
    sc, sq = _sc_call(tbl, u_idx, p_idx, n_idx)
    bpr, reg = _tc_epilogue(sc.reshape(B // 128, 128), sq)
    return (bpr.reshape(()), reg.reshape(()))


# speedup vs baseline: 1.5545x; 1.5545x over previous
import jax
import jax.numpy as jnp
from jax import lax
from jax.experimental import pallas as pl
from jax.experimental.pallas import tpu as pltpu
from jax.experimental.pallas import tpu_sc as plsc

N_USERS = 100000
N_ITEMS = 900000
EMB = 32
REGS = 1e-5
B = 16384

NC = 2
NS = 16
NW = NC * NS
PB = B // NW
CHUNK = 128
NCH = PB // CHUNK
QT = 262144
BPW = QT // NW
RCH = 128
NRC = BPW // RCH


def _repack_body(tbl_t_hbm, out_hbm, slab_a, slab_b, out_va, out_vb,
                 sem_in, sem_out):
    wid = lax.axis_index("s") * NC + lax.axis_index("c")
    base = wid * BPW

    def fetch(c, slab):
        copies = []
        for j in range(4):
            col = j * QT + base + c * RCH
            col = jnp.where(col + RCH > 1000064, 0, col)
            copies.append(pltpu.async_copy(
                tbl_t_hbm.at[:, pl.ds(col, RCH)], slab.at[j], sem_in))
        return copies

    pending = fetch(0, slab_a)
    for c in range(NRC):
        slab, nxt = (slab_a, slab_b) if c % 2 == 0 else (slab_b, slab_a)
        out_v = out_va if c % 2 == 0 else out_vb
        for cp in pending:
            cp.wait()
        if c + 1 < NRC:
            pending = fetch(c + 1, nxt)
        if c >= 2:
            pltpu.make_async_copy(
                out_v, out_hbm.at[pl.ds(base, RCH)], sem_out).wait()

        def row(mm, _):
            for j in range(4):
                for h in range(2):
                    seg = plsc.load_gather(
                        slab.at[j],
                        [lax.iota(jnp.int32, 16) + h * 16,
                         jnp.zeros((16,), jnp.int32) + mm])
                    out_v[mm, pl.ds(j * EMB + h * 16, 16)] = seg
            return 0

        lax.fori_loop(0, RCH, row, 0)
        pltpu.async_copy(out_v, out_hbm.at[pl.ds(base + c * RCH, RCH)],
                         sem_out)
    for out_v in ((out_va, out_vb) if NRC >= 2 else (out_va,)):
        pltpu.make_async_copy(
            out_v, out_hbm.at[pl.ds(base, RCH)], sem_out).wait()


def _repack(tbl_t):
    mesh = plsc.VectorSubcoreMesh(core_axis_name="c", subcore_axis_name="s")
    return pl.kernel(
        _repack_body,
        out_type=jax.ShapeDtypeStruct((QT, 128), jnp.float32),
        mesh=mesh,
        compiler_params=pltpu.CompilerParams(
            use_tc_tiling_on_sc=True, needs_layout_passes=False),
        scratch_types=[
            pltpu.VMEM((4, EMB, RCH), jnp.float32),
            pltpu.VMEM((4, EMB, RCH), jnp.float32),
            pltpu.VMEM((RCH, 128), jnp.float32),
            pltpu.VMEM((RCH, 128), jnp.float32),
            pltpu.SemaphoreType.DMA,
            pltpu.SemaphoreType.DMA,
        ],
    )(tbl_t)


def _sc_body(tbl_hbm, u_idx_hbm, p_idx_hbm, n_idx_hbm,
             sc_hbm, sq_hbm,
             u_idx_v, p_idx_v, n_idx_v,
             uo_v, po_v, no_v,
             u_blk, p_blk, n_blk,
             sc_v, sq_v, sem):
    wid = lax.axis_index("s") * NC + lax.axis_index("c")

    pltpu.sync_copy(u_idx_hbm.at[wid], u_idx_v)
    pltpu.sync_copy(p_idx_hbm.at[wid], p_idx_v)
    pltpu.sync_copy(n_idx_hbm.at[wid], n_idx_v)
    for src, off in ((u_idx_v, uo_v), (p_idx_v, po_v), (n_idx_v, no_v)):
        for j in range(NCH):
            for v in range(CHUNK // 16):
                x = src[j, pl.ds(v * 16, 16)]
                off[j, pl.ds(v * 16, 16)] = lax.shift_right_logical(x, 18) * 32
                src[j, pl.ds(v * 16, 16)] = x & (QT - 1)

    sq = jnp.zeros((16,), jnp.float32)
    for p in range(NCH):
        copies = []
        for idx_v, blk in ((u_idx_v, u_blk), (p_idx_v, p_blk), (n_idx_v, n_blk)):
            copies.append(pltpu.async_copy(
                tbl_hbm.at[idx_v.at[p]], blk, sem))
        for c in copies:
            c.wait()

        def group(g, sq):
            lanes = pl.ds(g * 16, 16)
            ku = lax.iota(jnp.int32, 16) + g * 16
            offu = uo_v[p, lanes]
            offp = po_v[p, lanes]
            offn = no_v[p, lanes]

            def dim(d, carry):
                acc, sq = carry
                vu = plsc.load_gather(u_blk, [ku, offu + d])
                vp = plsc.load_gather(p_blk, [ku, offp + d])
                vn = plsc.load_gather(n_blk, [ku, offn + d])
                return (acc + vu * (vp - vn),
                        sq + vu * vu + vp * vp + vn * vn)

            acc, sq = lax.fori_loop(
                0, EMB, dim, (jnp.zeros((16,), jnp.float32), sq))
            sc_v[p, lanes] = acc
            return sq

        sq = lax.fori_loop(0, CHUNK // 16, group, sq)

    sq_v[...] = sq
    pltpu.sync_copy(sc_v, sc_hbm.at[wid])
    pltpu.sync_copy(sq_v, sq_hbm.at[wid])


def _sc_call(tbl, u_idx, p_idx, n_idx):
    mesh = plsc.VectorSubcoreMesh(core_axis_name="c", subcore_axis_name="s")
    return pl.kernel(
        _sc_body,
        out_type=(
            jax.ShapeDtypeStruct((NW, NCH, CHUNK), jnp.float32),
            jax.ShapeDtypeStruct((NW, 16), jnp.float32),
        ),
        mesh=mesh,
        compiler_params=pltpu.CompilerParams(
            use_tc_tiling_on_sc=True, needs_layout_passes=False),
        scratch_types=[
            pltpu.VMEM((NCH, CHUNK), jnp.int32),
            pltpu.VMEM((NCH, CHUNK), jnp.int32),
            pltpu.VMEM((NCH, CHUNK), jnp.int32),
            pltpu.VMEM((NCH, CHUNK), jnp.int32),
            pltpu.VMEM((NCH, CHUNK), jnp.int32),
            pltpu.VMEM((NCH, CHUNK), jnp.int32),
            pltpu.VMEM((CHUNK, 128), jnp.float32),
            pltpu.VMEM((CHUNK, 128), jnp.float32),
            pltpu.VMEM((CHUNK, 128), jnp.float32),
            pltpu.VMEM((NCH, CHUNK), jnp.float32),
            pltpu.VMEM((16,), jnp.float32),
            pltpu.SemaphoreType.DMA,
        ],
    )(tbl, u_idx, p_idx, n_idx)


def _tc_epilogue_body(sc_ref, sq_ref, bpr_ref, reg_ref):
    d = sc_ref[...]
    logsig = -jnp.log1p(jnp.exp(-d))
    bpr_ref[...] = jnp.full((1, 1), -jnp.mean(logsig), jnp.float32)
    reg_ref[...] = jnp.full((1, 1), REGS * 0.5 * jnp.sum(sq_ref[...]),
                            jnp.float32)


def _tc_epilogue(sc, sq):
    return pl.pallas_call(
        _tc_epilogue_body,
        out_shape=(
            jax.ShapeDtypeStruct((1, 1), jnp.float32),
            jax.ShapeDtypeStruct((1, 1), jnp.float32),
        ),
    )(sc, sq)


@jax.jit
def kernel(user, pos_item, neg_item, table):
    tbl_t = table.T
    tbl = _repack(tbl_t)
    u_idx = user.reshape(NW, NCH, CHUNK)
    p_idx = pos_item.reshape(NW, NCH, CHUNK)
    n_idx = neg_item.reshape(NW, NCH, CHUNK)
    sc, sq = _sc_call(tbl, u_idx, p_idx, n_idx)
    bpr, reg = _tc_epilogue(sc.reshape(B // 128, 128), sq)
    return (bpr.reshape(()), reg.reshape(()))


# speedup vs baseline: 2.5207x; 1.6216x over previous
import jax
import jax.numpy as jnp
from jax import lax
from jax.experimental import pallas as pl
from jax.experimental.pallas import tpu as pltpu
from jax.experimental.pallas import tpu_sc as plsc

N_USERS = 100000
N_ITEMS = 900000
EMB = 32
REGS = 1e-5
B = 16384

NC = 2
NS = 16
NW = NC * NS
PB = B // NW
CHUNK = 128
NCH = PB // CHUNK


def _sc_body(table_hbm, u_idx_hbm, p_idx_hbm, n_idx_hbm,
             pd_hbm, sq_hbm,
             u_idx_v, p_idx_v, n_idx_v,
             u_rows, p_rows, n_rows,
             pd_v, sq_v, sem):
    wid = lax.axis_index("s") * NC + lax.axis_index("c")

    pltpu.sync_copy(u_idx_hbm.at[wid], u_idx_v)
    pltpu.sync_copy(p_idx_hbm.at[wid], p_idx_v)
    pltpu.sync_copy(n_idx_hbm.at[wid], n_idx_v)

    copies = []
    for idx_v, rows in ((u_idx_v, u_rows), (p_idx_v, p_rows), (n_idx_v, n_rows)):
        for j in range(NCH):
            copies.append(pltpu.async_copy(
                table_hbm.at[idx_v.at[j]],
                rows.at[pl.ds(j * CHUNK, CHUNK)],
                sem))
    for c in copies:
        c.wait()

    def body(i, sq):
        u0 = u_rows[i, pl.ds(0, 16)]
        u1 = u_rows[i, pl.ds(16, 16)]
        p0 = p_rows[i, pl.ds(0, 16)]
        p1 = p_rows[i, pl.ds(16, 16)]
        n0 = n_rows[i, pl.ds(0, 16)]
        n1 = n_rows[i, pl.ds(16, 16)]
        pd_v[i, :] = u0 * (p0 - n0) + u1 * (p1 - n1)
        return (sq + u0 * u0 + u1 * u1 + p0 * p0 + p1 * p1
                + n0 * n0 + n1 * n1)

    sq = lax.fori_loop(0, PB, body, jnp.zeros((16,), jnp.float32))
    sq_v[...] = sq

    pltpu.sync_copy(pd_v, pd_hbm.at[wid])
    pltpu.sync_copy(sq_v, sq_hbm.at[wid])


def _sc_call(table, u_idx, p_idx, n_idx):
    mesh = plsc.VectorSubcoreMesh(core_axis_name="c", subcore_axis_name="s")
    return pl.kernel(
        _sc_body,
        out_type=(
            jax.ShapeDtypeStruct((NW, PB, 16), jnp.float32),
            jax.ShapeDtypeStruct((NW, 16), jnp.float32),
        ),
        mesh=mesh,
        compiler_params=pltpu.CompilerParams(use_tc_tiling_on_sc=False),
        scratch_types=[
            pltpu.VMEM((NCH, CHUNK), jnp.int32),
            pltpu.VMEM((NCH, CHUNK), jnp.int32),
            pltpu.VMEM((NCH, CHUNK), jnp.int32),
            pltpu.VMEM((PB, EMB), jnp.float32),
            pltpu.VMEM((PB, EMB), jnp.float32),
            pltpu.VMEM((PB, EMB), jnp.float32),
            pltpu.VMEM((PB, 16), jnp.float32),
            pltpu.VMEM((16,), jnp.float32),
            pltpu.SemaphoreType.DMA,
        ],
    )(table, u_idx, p_idx, n_idx)


def _tc_epilogue_body(pd_ref, sq_ref, bpr_ref, reg_ref):
    d = jnp.sum(pd_ref[...], axis=1, keepdims=True)
    logsig = -jnp.log1p(jnp.exp(-d))
    bpr_ref[...] = jnp.full((1, 1), -jnp.mean(logsig), jnp.float32)
    reg_ref[...] = jnp.full((1, 1), REGS * 0.5 * jnp.sum(sq_ref[...]),
                            jnp.float32)


def _tc_epilogue(pd, sq):
    return pl.pallas_call(
        _tc_epilogue_body,
        out_shape=(
            jax.ShapeDtypeStruct((1, 1), jnp.float32),
            jax.ShapeDtypeStruct((1, 1), jnp.float32),
        ),
    )(pd, sq)


@jax.jit
def kernel(user, pos_item, neg_item, table):
    u_idx = user.reshape(NW, NCH, CHUNK)
    p_idx = pos_item.reshape(NW, NCH, CHUNK)
    n_idx = neg_item.reshape(NW, NCH, CHUNK)
    pd, sq = _sc_call(table, u_idx, p_idx, n_idx)
    bpr, reg = _tc_epilogue(pd.reshape(B, 16), sq)
    return (bpr.reshape(()), reg.reshape(()))
